# Initial kernel scaffold; baseline (speedup 1.0000x reference)
import jax
import jax.numpy as jnp
from jax.experimental import pallas as pl


def kernel(tokens, codes):
    raise NotImplementedError("write your pallas kernel here")


# speedup vs baseline: 9.2043x; 9.2043x over previous
import functools

import jax
import jax.numpy as jnp
from jax import lax
from jax.experimental import pallas as pl
from jax.experimental.pallas import tpu as pltpu
from jax.experimental.pallas import tpu_sc as plsc

V = 100000
D = 128
B = 4096 * 200
NC = 2
NS = 16
NW = NC * NS
BPW = B // NW
CH = 128
NBUF = 4
NCH = BPW // CH

assert NCH % NBUF == 0

_mesh = plsc.VectorSubcoreMesh(core_axis_name="c", subcore_axis_name="s")


@functools.partial(
    pl.kernel,
    mesh=_mesh,
    out_type=jax.ShapeDtypeStruct((B, D), jnp.float32),
    scratch_types=(
        [pltpu.VMEM((BPW,), jnp.int32)]
        + [pltpu.VMEM((CH, D), jnp.float32) for _ in range(NBUF)]
        + [pltpu.SemaphoreType.DMA for _ in range(NBUF)]
    ),
)
def _sc_gather(tok_hbm, codes_hbm, out_hbm, idx_v, *bufs_sems):
    bufs = bufs_sems[:NBUF]
    sems = bufs_sems[NBUF:]
    wid = lax.axis_index("s") * NC + lax.axis_index("c")
    base = wid * BPW

    pltpu.sync_copy(tok_hbm.at[pl.ds(base, BPW)], idx_v)

    for b in range(NBUF):
        pltpu.async_copy(
            codes_hbm.at[idx_v.at[pl.ds(b * CH, CH)]], bufs[b], sems[b]
        )

    def group(gi, carry):
        c0 = gi * NBUF
        for b in range(NBUF):
            c = c0 + b
            pltpu.make_async_copy(
                codes_hbm.at[idx_v.at[pl.ds(c * CH, CH)]], bufs[b], sems[b]
            ).wait()
            pltpu.sync_copy(bufs[b], out_hbm.at[pl.ds(base + c * CH, CH)])
            nxt = c + NBUF

            @pl.when(nxt < NCH)
            def _():
                pltpu.async_copy(
                    codes_hbm.at[idx_v.at[pl.ds(nxt * CH, CH)]], bufs[b], sems[b]
                )

        return carry

    lax.fori_loop(0, NCH // NBUF, group, 0)


def kernel(tokens, codes):
    idx = tokens.reshape(-1).astype(jnp.int32)
    out = _sc_gather(idx, codes)
    return out.reshape(tokens.shape + (D,))
